# baseline (speedup 1.0000x reference)
import functools

import jax
import jax.numpy as jnp
from jax import lax
from jax.experimental import pallas as pl
from jax.experimental.pallas import tpu as pltpu
from jax.experimental.pallas import tpu_sc as plsc

N_NODES = 10000
N_EDGES = 160000
TWO_E = 2 * N_EDGES
D_FEAT = 256
HALF = 8
ROW_W = 16
LANES = 16

NUM_CORES = 2
NUM_SUBCORES = 16
NUM_WORKERS = NUM_CORES * NUM_SUBCORES
BLK = 125
GRP = 8

CT = N_EDGES // 128
CT_BASE = CT // NUM_WORKERS
CT_MAX = CT_BASE + 1

BPW_B = TWO_E // (NUM_WORKERS * BLK)
NBLK_B = TWO_E // BLK
NGRP_B = BPW_B // GRP

NP = N_NODES
RPT = NP // NUM_SUBCORES


def _worker_id():
    return lax.axis_index("c") * NUM_SUBCORES + lax.axis_index("s")


_MESH = plsc.VectorSubcoreMesh(core_axis_name="c", subcore_axis_name="s")
_SC_PARAMS = pltpu.CompilerParams(use_tc_tiling_on_sc=False,
                                  needs_layout_passes=False)


@functools.partial(
    pl.kernel,
    out_type=jax.ShapeDtypeStruct((NUM_CORES, 2, NP, ROW_W), jnp.float32),
    mesh=_MESH,
    scratch_types=[
        pltpu.VMEM((CT_MAX, 128), jnp.int32),
        pltpu.VMEM((CT_MAX, 128), jnp.int32),
        pltpu.VMEM((ROW_W, 128), jnp.float32),
        pltpu.VMEM((128, ROW_W), jnp.float32),
        pltpu.VMEM((RPT, ROW_W), jnp.float32),
        pltpu.VMEM_SHARED((NP, ROW_W), jnp.float32),
        pltpu.VMEM_SHARED((NP, ROW_W), jnp.float32),
        pltpu.SemaphoreType.DMA,
    ],
    compiler_params=_SC_PARAMS,
)
def _scatter_edges(ea4_hbm, sidx_hbm, ridx_hbm, zeros_hbm, out_hbm,
                   sidx_v, ridx_v, fmbuf, rowbuf, tmp,
                   accs_sh, accr_sh, sem_st):
    cid = lax.axis_index("c")
    sid = lax.axis_index("s")
    w = _worker_id()
    z0 = sid * RPT
    pltpu.sync_copy(zeros_hbm.at[pl.ds(z0, RPT)], accs_sh.at[pl.ds(z0, RPT)])
    pltpu.sync_copy(zeros_hbm.at[pl.ds(z0, RPT)], accr_sh.at[pl.ds(z0, RPT)])
    nct = CT_BASE + jnp.where(w < CT % NUM_WORKERS, 1, 0)
    base = CT_BASE * w + jnp.minimum(w, CT % NUM_WORKERS)
    pltpu.sync_copy(sidx_hbm.at[pl.ds(base, CT_BASE)],
                    sidx_v.at[pl.ds(0, CT_BASE)])
    pltpu.sync_copy(ridx_hbm.at[pl.ds(base, CT_BASE)],
                    ridx_v.at[pl.ds(0, CT_BASE)])

    @pl.when(w < CT % NUM_WORKERS)
    def _():
        pltpu.sync_copy(sidx_hbm.at[pl.ds(base + CT_BASE, 1)],
                        sidx_v.at[pl.ds(CT_BASE, 1)])
        pltpu.sync_copy(ridx_hbm.at[pl.ds(base + CT_BASE, 1)],
                        ridx_v.at[pl.ds(CT_BASE, 1)])

    plsc.subcore_barrier()
    lane = lax.iota(jnp.int32, LANES)

    def coltile(j, carry):
        ct = base + j
        pltpu.sync_copy(ea4_hbm.at[0, ct], fmbuf.at[pl.ds(0, HALF)])
        pltpu.sync_copy(ea4_hbm.at[1, ct], fmbuf.at[pl.ds(HALF, HALF)])

        def transpose(e, c2):
            rowbuf[e] = plsc.load_gather(
                fmbuf, [lane, jnp.full((LANES,), e, dtype=jnp.int32)])
            return c2

        lax.fori_loop(0, 128, transpose, 0)
        d1 = pltpu.async_copy(rowbuf, accs_sh.at[sidx_v.at[j]],
                              sem_st, add=True)
        d2 = pltpu.async_copy(rowbuf, accr_sh.at[ridx_v.at[j]],
                              sem_st, add=True)
        d1.wait()
        d2.wait()
        return carry

    lax.fori_loop(0, nct, coltile, 0)
    plsc.subcore_barrier()
    pltpu.sync_copy(accs_sh.at[pl.ds(z0, RPT)], tmp)
    pltpu.sync_copy(tmp, out_hbm.at[cid, 0, pl.ds(z0, RPT)])
    pltpu.sync_copy(accr_sh.at[pl.ds(z0, RPT)], tmp)
    pltpu.sync_copy(tmp, out_hbm.at[cid, 1, pl.ds(z0, RPT)])


@functools.partial(
    pl.kernel,
    out_type=jax.ShapeDtypeStruct((NUM_CORES, NP, ROW_W), jnp.float32),
    mesh=_MESH,
    scratch_types=[
        pltpu.VMEM((BPW_B, BLK), jnp.int32),
        pltpu.VMEM((BPW_B, BLK), jnp.int32),
        pltpu.VMEM((GRP * BLK, ROW_W), jnp.float32),
        pltpu.VMEM((RPT, ROW_W), jnp.float32),
        pltpu.VMEM((RPT, ROW_W), jnp.float32),
        pltpu.VMEM((RPT, ROW_W), jnp.float32),
        pltpu.VMEM((RPT, ROW_W), jnp.float32),
        pltpu.VMEM((RPT, ROW_W), jnp.float32),
        pltpu.VMEM_SHARED((NP, ROW_W), jnp.float32),
        pltpu.VMEM_SHARED((NP, ROW_W), jnp.float32),
        pltpu.SemaphoreType.DMA,
        pltpu.SemaphoreType.DMA,
    ],
    compiler_params=_SC_PARAMS,
)
def _gather_scatter_add(acc_hbm, idxo_hbm, idxi_hbm, zeros_hbm, out_hbm,
                        idxo_v, idxi_v, gbuf, bs0, bs1, br0, br1, aggbuf,
                        agg_sh, sums_sh, sem_g, sem_s):
    cid = lax.axis_index("c")
    sid = lax.axis_index("s")
    w = _worker_id()
    z0 = sid * RPT
    pltpu.sync_copy(zeros_hbm.at[pl.ds(z0, RPT)], sums_sh.at[pl.ds(z0, RPT)])
    pltpu.sync_copy(idxo_hbm.at[pl.ds(w * BPW_B, BPW_B)], idxo_v)
    pltpu.sync_copy(idxi_hbm.at[pl.ds(w * BPW_B, BPW_B)], idxi_v)
    pltpu.sync_copy(acc_hbm.at[0, 0, pl.ds(z0, RPT)], bs0)
    pltpu.sync_copy(acc_hbm.at[1, 0, pl.ds(z0, RPT)], bs1)
    pltpu.sync_copy(acc_hbm.at[0, 1, pl.ds(z0, RPT)], br0)
    pltpu.sync_copy(acc_hbm.at[1, 1, pl.ds(z0, RPT)], br1)
    lane = lax.iota(jnp.int32, LANES)
    perm = lane ^ HALF
    tail = jnp.where(lane == HALF, 1.0, 0.0)
    lo = lane < HALF

    def comb(i, carry):
        srow = bs0[i] + bs1[i]
        spl = jnp.full((LANES,), i, dtype=jnp.int32)
        rrot = (plsc.load_gather(br0, [spl, perm])
                + plsc.load_gather(br1, [spl, perm]))
        aggbuf[i] = jnp.where(lo, srow + rrot, tail)
        return carry

    lax.fori_loop(0, RPT, comb, 0)
    pltpu.sync_copy(aggbuf, agg_sh.at[pl.ds(z0, RPT)])
    plsc.subcore_barrier()

    def group(g, carry):
        descs = [
            pltpu.async_copy(agg_sh.at[idxo_v.at[g * GRP + bq]],
                             gbuf.at[pl.ds(bq * BLK, BLK)], sem_g)
            for bq in range(GRP)
        ]
        for d in descs:
            d.wait()
        descs = [
            pltpu.async_copy(gbuf.at[pl.ds(bq * BLK, BLK)],
                             sums_sh.at[idxi_v.at[g * GRP + bq]],
                             sem_s, add=True)
            for bq in range(GRP)
        ]
        for d in descs:
            d.wait()
        return carry

    lax.fori_loop(0, NGRP_B, group, 0)
    plsc.subcore_barrier()
    pltpu.sync_copy(sums_sh.at[pl.ds(z0, RPT)], tmp := bs0)
    pltpu.sync_copy(tmp, out_hbm.at[cid, pl.ds(z0, RPT)])


ROWS_BLK = 400


def _dense_body(x_ref, w_ref, b_ref, out_ref):
    out_ref[...] = jnp.dot(
        x_ref[...], w_ref[HALF:, :],
        preferred_element_type=jnp.float32) + b_ref[...]


def _finish_body(p1_ref, s0_ref, s1_ref, w_ref, out_ref):
    sums = s0_ref[0] + s1_ref[0]
    cnt = jnp.maximum(sums[:, HALF:HALF + 1], 1.0)
    navg = sums[:, :HALF] / cnt
    out_ref[...] = p1_ref[...] + jnp.dot(
        navg, w_ref[:HALF, :], preferred_element_type=jnp.float32)


def kernel(x, edge_attr, W, b, edge_index):
    senders = edge_index[0]
    receivers = edge_index[1]
    sidx = senders.reshape(CT, 128)
    ridx = receivers.reshape(CT, 128)
    ea4 = edge_attr.T.reshape(2, HALF, CT, 128).transpose(0, 2, 1, 3)
    zeros_np = jnp.zeros((NP, ROW_W), dtype=jnp.float32)

    grid = N_NODES // ROWS_BLK
    part1 = pl.pallas_call(
        _dense_body,
        grid=(grid,),
        in_specs=[
            pl.BlockSpec((ROWS_BLK, D_FEAT), lambda i: (i, 0)),
            pl.BlockSpec((D_FEAT + HALF, D_FEAT), lambda i: (0, 0)),
            pl.BlockSpec((1, D_FEAT), lambda i: (0, 0)),
        ],
        out_specs=pl.BlockSpec((ROWS_BLK, D_FEAT), lambda i: (i, 0)),
        out_shape=jax.ShapeDtypeStruct((N_NODES, D_FEAT), jnp.float32),
    )(x, W, b.reshape(1, D_FEAT))

    acc = _scatter_edges(ea4, sidx, ridx, zeros_np)

    idx_in = jnp.concatenate([senders, receivers]).reshape(NBLK_B, BLK)
    idx_out = jnp.concatenate([receivers, senders]).reshape(NBLK_B, BLK)
    sums_pair = _gather_scatter_add(acc, idx_out, idx_in, zeros_np)

    out = pl.pallas_call(
        _finish_body,
        grid=(grid,),
        in_specs=[
            pl.BlockSpec((ROWS_BLK, D_FEAT), lambda i: (i, 0)),
            pl.BlockSpec((1, ROWS_BLK, ROW_W), lambda i: (0, i, 0)),
            pl.BlockSpec((1, ROWS_BLK, ROW_W), lambda i: (1, i, 0)),
            pl.BlockSpec((D_FEAT + HALF, D_FEAT), lambda i: (0, 0)),
        ],
        out_specs=pl.BlockSpec((ROWS_BLK, D_FEAT), lambda i: (i, 0)),
        out_shape=jax.ShapeDtypeStruct((N_NODES, D_FEAT), jnp.float32),
    )(part1, sums_pair, sums_pair, W)
    return out


# speedup vs baseline: 1.3208x; 1.3208x over previous
import functools

import jax
import jax.numpy as jnp
from jax import lax
from jax.experimental import pallas as pl
from jax.experimental.pallas import tpu as pltpu
from jax.experimental.pallas import tpu_sc as plsc

N_NODES = 10000
N_EDGES = 160000
TWO_E = 2 * N_EDGES
D_FEAT = 256
HALF = 8
ROW_W = 16
LANES = 16

NUM_CORES = 2
NUM_SUBCORES = 16
NUM_WORKERS = NUM_CORES * NUM_SUBCORES
BLK = 125
GRP = 8

CT = N_EDGES // 128
CT_BASE = CT // NUM_WORKERS
CT_MAX = CT_BASE + 1

BPW_B = TWO_E // (NUM_WORKERS * BLK)
NBLK_B = TWO_E // BLK
NGRP_B = BPW_B // GRP

NP = N_NODES
RPT = NP // NUM_SUBCORES


def _worker_id():
    return lax.axis_index("c") * NUM_SUBCORES + lax.axis_index("s")


_MESH = plsc.VectorSubcoreMesh(core_axis_name="c", subcore_axis_name="s")
_SC_PARAMS = pltpu.CompilerParams(use_tc_tiling_on_sc=False,
                                  needs_layout_passes=False)


@functools.partial(
    pl.kernel,
    out_type=jax.ShapeDtypeStruct((NUM_CORES, 2, NP, ROW_W), jnp.float32),
    mesh=_MESH,
    scratch_types=[
        pltpu.VMEM((CT_MAX, 128), jnp.int32),
        pltpu.VMEM((CT_MAX, 128), jnp.int32),
        pltpu.VMEM((2, CT_MAX, HALF, 128), jnp.float32),
        pltpu.VMEM((2 * 128, ROW_W), jnp.float32),
        pltpu.VMEM((RPT, ROW_W), jnp.float32),
        pltpu.VMEM_SHARED((NP, ROW_W), jnp.float32),
        pltpu.VMEM_SHARED((NP, ROW_W), jnp.float32),
        pltpu.SemaphoreType.DMA,
        pltpu.SemaphoreType.DMA,
    ],
    compiler_params=_SC_PARAMS,
)
def _scatter_edges(ea4_hbm, sidx_hbm, ridx_hbm, zeros_hbm, out_hbm,
                   sidx_v, ridx_v, fmbuf, rowbuf, tmp,
                   accs_sh, accr_sh, sem_ld, sem_st):
    cid = lax.axis_index("c")
    sid = lax.axis_index("s")
    w = _worker_id()
    z0 = sid * RPT
    nct = CT_BASE + jnp.where(w < CT % NUM_WORKERS, 1, 0)
    base = CT_BASE * w + jnp.minimum(w, CT % NUM_WORKERS)
    loads = [
        pltpu.async_copy(ea4_hbm.at[0, pl.ds(base, CT_BASE)],
                         fmbuf.at[0, pl.ds(0, CT_BASE)], sem_ld),
        pltpu.async_copy(ea4_hbm.at[1, pl.ds(base, CT_BASE)],
                         fmbuf.at[1, pl.ds(0, CT_BASE)], sem_ld),
        pltpu.async_copy(sidx_hbm.at[pl.ds(base, CT_BASE)],
                         sidx_v.at[pl.ds(0, CT_BASE)], sem_ld),
        pltpu.async_copy(ridx_hbm.at[pl.ds(base, CT_BASE)],
                         ridx_v.at[pl.ds(0, CT_BASE)], sem_ld),
    ]
    pltpu.sync_copy(zeros_hbm.at[pl.ds(z0, RPT)], accs_sh.at[pl.ds(z0, RPT)])
    pltpu.sync_copy(zeros_hbm.at[pl.ds(z0, RPT)], accr_sh.at[pl.ds(z0, RPT)])

    @pl.when(w < CT % NUM_WORKERS)
    def _():
        pltpu.sync_copy(ea4_hbm.at[0, pl.ds(base + CT_BASE, 1)],
                        fmbuf.at[0, pl.ds(CT_BASE, 1)])
        pltpu.sync_copy(ea4_hbm.at[1, pl.ds(base + CT_BASE, 1)],
                        fmbuf.at[1, pl.ds(CT_BASE, 1)])
        pltpu.sync_copy(sidx_hbm.at[pl.ds(base + CT_BASE, 1)],
                        sidx_v.at[pl.ds(CT_BASE, 1)])
        pltpu.sync_copy(ridx_hbm.at[pl.ds(base + CT_BASE, 1)],
                        ridx_v.at[pl.ds(CT_BASE, 1)])

    for d in loads:
        d.wait()
    plsc.subcore_barrier()

    lane = lax.iota(jnp.int32, LANES)
    tvec = jnp.where(lane < HALF, 0, 1)
    rvec = lane & (HALF - 1)

    def coltile(j, carry):
        off = (j & 1) * 128
        @pl.when(j >= 2)
        def _():
            pltpu.make_async_copy(zeros_hbm.at[pl.ds(0, 128)],
                                  rowbuf.at[pl.ds(off, 128)], sem_st).wait()
            pltpu.make_async_copy(zeros_hbm.at[pl.ds(0, 128)],
                                  rowbuf.at[pl.ds(off, 128)], sem_st).wait()

        jvec = jnp.full((LANES,), j, dtype=jnp.int32)

        def transpose16(eh, c2):
            ebase = jnp.full((LANES,), eh * LANES, dtype=jnp.int32)
            for el in range(LANES):
                evec = ebase + el
                rowbuf[off + eh * LANES + el] = plsc.load_gather(
                    fmbuf, [tvec, jvec, rvec, evec])
            return c2

        lax.fori_loop(0, 128 // LANES, transpose16, 0)
        src = rowbuf.at[pl.ds(off, 128)]
        pltpu.async_copy(src, accs_sh.at[sidx_v.at[j]], sem_st, add=True)
        pltpu.async_copy(src, accr_sh.at[ridx_v.at[j]], sem_st, add=True)
        return carry

    lax.fori_loop(0, nct, coltile, 0)
    for _ in range(4):
        pltpu.make_async_copy(zeros_hbm.at[pl.ds(0, 128)],
                              rowbuf.at[pl.ds(0, 128)], sem_st).wait()
    plsc.subcore_barrier()
    pltpu.sync_copy(accs_sh.at[pl.ds(z0, RPT)], tmp)
    pltpu.sync_copy(tmp, out_hbm.at[cid, 0, pl.ds(z0, RPT)])
    pltpu.sync_copy(accr_sh.at[pl.ds(z0, RPT)], tmp)
    pltpu.sync_copy(tmp, out_hbm.at[cid, 1, pl.ds(z0, RPT)])


@functools.partial(
    pl.kernel,
    out_type=jax.ShapeDtypeStruct((NUM_CORES, NP, ROW_W), jnp.float32),
    mesh=_MESH,
    scratch_types=[
        pltpu.VMEM((BPW_B, BLK), jnp.int32),
        pltpu.VMEM((BPW_B, BLK), jnp.int32),
        pltpu.VMEM((GRP * BLK, ROW_W), jnp.float32),
        pltpu.VMEM((RPT, ROW_W), jnp.float32),
        pltpu.VMEM((RPT, ROW_W), jnp.float32),
        pltpu.VMEM((RPT, ROW_W), jnp.float32),
        pltpu.VMEM((RPT, ROW_W), jnp.float32),
        pltpu.VMEM((RPT, ROW_W), jnp.float32),
        pltpu.VMEM_SHARED((NP, ROW_W), jnp.float32),
        pltpu.VMEM_SHARED((NP, ROW_W), jnp.float32),
        pltpu.SemaphoreType.DMA,
        pltpu.SemaphoreType.DMA,
    ],
    compiler_params=_SC_PARAMS,
)
def _gather_scatter_add(acc_hbm, idxo_hbm, idxi_hbm, zeros_hbm, out_hbm,
                        idxo_v, idxi_v, gbuf, bs0, bs1, br0, br1, aggbuf,
                        agg_sh, sums_sh, sem_g, sem_s):
    cid = lax.axis_index("c")
    sid = lax.axis_index("s")
    w = _worker_id()
    z0 = sid * RPT
    pltpu.sync_copy(zeros_hbm.at[pl.ds(z0, RPT)], sums_sh.at[pl.ds(z0, RPT)])
    pltpu.sync_copy(idxo_hbm.at[pl.ds(w * BPW_B, BPW_B)], idxo_v)
    pltpu.sync_copy(idxi_hbm.at[pl.ds(w * BPW_B, BPW_B)], idxi_v)
    pltpu.sync_copy(acc_hbm.at[0, 0, pl.ds(z0, RPT)], bs0)
    pltpu.sync_copy(acc_hbm.at[1, 0, pl.ds(z0, RPT)], bs1)
    pltpu.sync_copy(acc_hbm.at[0, 1, pl.ds(z0, RPT)], br0)
    pltpu.sync_copy(acc_hbm.at[1, 1, pl.ds(z0, RPT)], br1)
    lane = lax.iota(jnp.int32, LANES)
    perm = lane ^ HALF
    tail = jnp.where(lane == HALF, 1.0, 0.0)
    lo = lane < HALF

    def comb(i, carry):
        srow = bs0[i] + bs1[i]
        spl = jnp.full((LANES,), i, dtype=jnp.int32)
        rrot = (plsc.load_gather(br0, [spl, perm])
                + plsc.load_gather(br1, [spl, perm]))
        aggbuf[i] = jnp.where(lo, srow + rrot, tail)
        return carry

    lax.fori_loop(0, RPT, comb, 0)
    pltpu.sync_copy(aggbuf, agg_sh.at[pl.ds(z0, RPT)])
    plsc.subcore_barrier()

    def group(g, carry):
        descs = [
            pltpu.async_copy(agg_sh.at[idxo_v.at[g * GRP + bq]],
                             gbuf.at[pl.ds(bq * BLK, BLK)], sem_g)
            for bq in range(GRP)
        ]
        for d in descs:
            d.wait()
        descs = [
            pltpu.async_copy(gbuf.at[pl.ds(bq * BLK, BLK)],
                             sums_sh.at[idxi_v.at[g * GRP + bq]],
                             sem_s, add=True)
            for bq in range(GRP)
        ]
        for d in descs:
            d.wait()
        return carry

    lax.fori_loop(0, NGRP_B, group, 0)
    plsc.subcore_barrier()
    pltpu.sync_copy(sums_sh.at[pl.ds(z0, RPT)], tmp := bs0)
    pltpu.sync_copy(tmp, out_hbm.at[cid, pl.ds(z0, RPT)])


ROWS_BLK = 400


def _dense_body(x_ref, w_ref, b_ref, out_ref):
    out_ref[...] = jnp.dot(
        x_ref[...], w_ref[HALF:, :],
        preferred_element_type=jnp.float32) + b_ref[...]


def _finish_body(p1_ref, s0_ref, s1_ref, w_ref, out_ref):
    sums = s0_ref[0] + s1_ref[0]
    cnt = jnp.maximum(sums[:, HALF:HALF + 1], 1.0)
    navg = sums[:, :HALF] / cnt
    out_ref[...] = p1_ref[...] + jnp.dot(
        navg, w_ref[:HALF, :], preferred_element_type=jnp.float32)


def kernel(x, edge_attr, W, b, edge_index):
    senders = edge_index[0]
    receivers = edge_index[1]
    sidx = senders.reshape(CT, 128)
    ridx = receivers.reshape(CT, 128)
    ea4 = edge_attr.T.reshape(2, HALF, CT, 128).transpose(0, 2, 1, 3)
    zeros_np = jnp.zeros((NP, ROW_W), dtype=jnp.float32)

    grid = N_NODES // ROWS_BLK
    part1 = pl.pallas_call(
        _dense_body,
        grid=(grid,),
        in_specs=[
            pl.BlockSpec((ROWS_BLK, D_FEAT), lambda i: (i, 0)),
            pl.BlockSpec((D_FEAT + HALF, D_FEAT), lambda i: (0, 0)),
            pl.BlockSpec((1, D_FEAT), lambda i: (0, 0)),
        ],
        out_specs=pl.BlockSpec((ROWS_BLK, D_FEAT), lambda i: (i, 0)),
        out_shape=jax.ShapeDtypeStruct((N_NODES, D_FEAT), jnp.float32),
    )(x, W, b.reshape(1, D_FEAT))

    acc = _scatter_edges(ea4, sidx, ridx, zeros_np)

    idx_in = jnp.concatenate([senders, receivers]).reshape(NBLK_B, BLK)
    idx_out = jnp.concatenate([receivers, senders]).reshape(NBLK_B, BLK)
    sums_pair = _gather_scatter_add(acc, idx_out, idx_in, zeros_np)

    out = pl.pallas_call(
        _finish_body,
        grid=(grid,),
        in_specs=[
            pl.BlockSpec((ROWS_BLK, D_FEAT), lambda i: (i, 0)),
            pl.BlockSpec((1, ROWS_BLK, ROW_W), lambda i: (0, i, 0)),
            pl.BlockSpec((1, ROWS_BLK, ROW_W), lambda i: (1, i, 0)),
            pl.BlockSpec((D_FEAT + HALF, D_FEAT), lambda i: (0, 0)),
        ],
        out_specs=pl.BlockSpec((ROWS_BLK, D_FEAT), lambda i: (i, 0)),
        out_shape=jax.ShapeDtypeStruct((N_NODES, D_FEAT), jnp.float32),
    )(part1, sums_pair, sums_pair, W)
    return out


# speedup vs baseline: 1.5646x; 1.1846x over previous
import functools

import jax
import jax.numpy as jnp
from jax import lax
from jax.experimental import pallas as pl
from jax.experimental.pallas import tpu as pltpu
from jax.experimental.pallas import tpu_sc as plsc

N_NODES = 10000
N_EDGES = 160000
TWO_E = 2 * N_EDGES
D_FEAT = 256
HALF = 8
ROW_W = 16
LANES = 16

NUM_CORES = 2
NUM_SUBCORES = 16
NUM_WORKERS = NUM_CORES * NUM_SUBCORES
BLK = 125
GRP = 8

CT = N_EDGES // 128
CT_BASE = CT // NUM_WORKERS
CT_MAX = CT_BASE + 1

BPW_B = TWO_E // (NUM_WORKERS * BLK)
NBLK_B = TWO_E // BLK
NGRP_B = BPW_B // GRP

NP = N_NODES
RPT = NP // NUM_SUBCORES


def _worker_id():
    return lax.axis_index("c") * NUM_SUBCORES + lax.axis_index("s")


_MESH = plsc.VectorSubcoreMesh(core_axis_name="c", subcore_axis_name="s")
_SC_PARAMS = pltpu.CompilerParams(use_tc_tiling_on_sc=False,
                                  needs_layout_passes=False)


@functools.partial(
    pl.kernel,
    out_type=jax.ShapeDtypeStruct((NUM_CORES, 2, NP, ROW_W), jnp.float32),
    mesh=_MESH,
    scratch_types=[
        pltpu.VMEM((CT_MAX, 128), jnp.int32),
        pltpu.VMEM((CT_MAX, 128), jnp.int32),
        pltpu.VMEM((2, CT_MAX, HALF, 128), jnp.float32),
        pltpu.VMEM((2 * 128, ROW_W), jnp.float32),
        pltpu.VMEM((RPT, ROW_W), jnp.float32),
        pltpu.VMEM_SHARED((NP, ROW_W), jnp.float32),
        pltpu.VMEM_SHARED((NP, ROW_W), jnp.float32),
        pltpu.SemaphoreType.DMA,
        pltpu.SemaphoreType.DMA,
    ],
    compiler_params=_SC_PARAMS,
)
def _scatter_edges(ea4_hbm, sidx_hbm, ridx_hbm, zeros_hbm, out_hbm,
                   sidx_v, ridx_v, fmbuf, rowbuf, tmp,
                   accs_sh, accr_sh, sem_ld, sem_st):
    cid = lax.axis_index("c")
    sid = lax.axis_index("s")
    w = _worker_id()
    z0 = sid * RPT
    nct = CT_BASE + jnp.where(w < CT % NUM_WORKERS, 1, 0)
    base = CT_BASE * w + jnp.minimum(w, CT % NUM_WORKERS)
    loads = [
        pltpu.async_copy(ea4_hbm.at[0, pl.ds(base, CT_BASE)],
                         fmbuf.at[0, pl.ds(0, CT_BASE)], sem_ld),
        pltpu.async_copy(ea4_hbm.at[1, pl.ds(base, CT_BASE)],
                         fmbuf.at[1, pl.ds(0, CT_BASE)], sem_ld),
        pltpu.async_copy(sidx_hbm.at[pl.ds(base, CT_BASE)],
                         sidx_v.at[pl.ds(0, CT_BASE)], sem_ld),
        pltpu.async_copy(ridx_hbm.at[pl.ds(base, CT_BASE)],
                         ridx_v.at[pl.ds(0, CT_BASE)], sem_ld),
    ]
    pltpu.sync_copy(zeros_hbm.at[pl.ds(z0, RPT)], accs_sh.at[pl.ds(z0, RPT)])
    pltpu.sync_copy(zeros_hbm.at[pl.ds(z0, RPT)], accr_sh.at[pl.ds(z0, RPT)])

    @pl.when(w < CT % NUM_WORKERS)
    def _():
        pltpu.sync_copy(ea4_hbm.at[0, pl.ds(base + CT_BASE, 1)],
                        fmbuf.at[0, pl.ds(CT_BASE, 1)])
        pltpu.sync_copy(ea4_hbm.at[1, pl.ds(base + CT_BASE, 1)],
                        fmbuf.at[1, pl.ds(CT_BASE, 1)])
        pltpu.sync_copy(sidx_hbm.at[pl.ds(base + CT_BASE, 1)],
                        sidx_v.at[pl.ds(CT_BASE, 1)])
        pltpu.sync_copy(ridx_hbm.at[pl.ds(base + CT_BASE, 1)],
                        ridx_v.at[pl.ds(CT_BASE, 1)])

    for d in loads:
        d.wait()
    plsc.subcore_barrier()

    lane = lax.iota(jnp.int32, LANES)
    tvec = jnp.where(lane < HALF, 0, 1)
    rvec = lane & (HALF - 1)

    def coltile(j, carry):
        off = (j & 1) * 128
        @pl.when(j >= 2)
        def _():
            pltpu.make_async_copy(zeros_hbm.at[pl.ds(0, 128)],
                                  rowbuf.at[pl.ds(off, 128)], sem_st).wait()
            pltpu.make_async_copy(zeros_hbm.at[pl.ds(0, 128)],
                                  rowbuf.at[pl.ds(off, 128)], sem_st).wait()

        jvec = jnp.full((LANES,), j, dtype=jnp.int32)

        def transpose16(eh, c2):
            ebase = jnp.full((LANES,), eh * LANES, dtype=jnp.int32)
            vals = [plsc.load_gather(fmbuf, [tvec, jvec, rvec, ebase + el])
                    for el in range(LANES)]
            for el in range(LANES):
                rowbuf[off + eh * LANES + el] = vals[el]
            return c2

        lax.fori_loop(0, 128 // LANES, transpose16, 0)
        src = rowbuf.at[pl.ds(off, 128)]
        pltpu.async_copy(src, accs_sh.at[sidx_v.at[j]], sem_st, add=True)
        pltpu.async_copy(src, accr_sh.at[ridx_v.at[j]], sem_st, add=True)
        return carry

    lax.fori_loop(0, nct, coltile, 0)
    for _ in range(4):
        pltpu.make_async_copy(zeros_hbm.at[pl.ds(0, 128)],
                              rowbuf.at[pl.ds(0, 128)], sem_st).wait()
    plsc.subcore_barrier()
    pltpu.sync_copy(accs_sh.at[pl.ds(z0, RPT)], tmp)
    pltpu.sync_copy(tmp, out_hbm.at[cid, 0, pl.ds(z0, RPT)])
    pltpu.sync_copy(accr_sh.at[pl.ds(z0, RPT)], tmp)
    pltpu.sync_copy(tmp, out_hbm.at[cid, 1, pl.ds(z0, RPT)])


@functools.partial(
    pl.kernel,
    out_type=jax.ShapeDtypeStruct((NUM_CORES, NP, ROW_W), jnp.float32),
    mesh=_MESH,
    scratch_types=[
        pltpu.VMEM((BPW_B, BLK), jnp.int32),
        pltpu.VMEM((BPW_B, BLK), jnp.int32),
        pltpu.VMEM((GRP * BLK, ROW_W), jnp.float32),
        pltpu.VMEM((RPT, ROW_W), jnp.float32),
        pltpu.VMEM((RPT, ROW_W), jnp.float32),
        pltpu.VMEM((RPT, ROW_W), jnp.float32),
        pltpu.VMEM((RPT, ROW_W), jnp.float32),
        pltpu.VMEM((RPT, ROW_W), jnp.float32),
        pltpu.VMEM_SHARED((NP, ROW_W), jnp.float32),
        pltpu.VMEM_SHARED((NP, ROW_W), jnp.float32),
        pltpu.SemaphoreType.DMA,
        pltpu.SemaphoreType.DMA,
    ],
    compiler_params=_SC_PARAMS,
)
def _gather_scatter_add(acc_hbm, idxo_hbm, idxi_hbm, zeros_hbm, out_hbm,
                        idxo_v, idxi_v, gbuf, bs0, bs1, br0, br1, aggbuf,
                        agg_sh, sums_sh, sem_g, sem_s):
    cid = lax.axis_index("c")
    sid = lax.axis_index("s")
    w = _worker_id()
    z0 = sid * RPT
    pltpu.sync_copy(zeros_hbm.at[pl.ds(z0, RPT)], sums_sh.at[pl.ds(z0, RPT)])
    pltpu.sync_copy(idxo_hbm.at[pl.ds(w * BPW_B, BPW_B)], idxo_v)
    pltpu.sync_copy(idxi_hbm.at[pl.ds(w * BPW_B, BPW_B)], idxi_v)
    pltpu.sync_copy(acc_hbm.at[0, 0, pl.ds(z0, RPT)], bs0)
    pltpu.sync_copy(acc_hbm.at[1, 0, pl.ds(z0, RPT)], bs1)
    pltpu.sync_copy(acc_hbm.at[0, 1, pl.ds(z0, RPT)], br0)
    pltpu.sync_copy(acc_hbm.at[1, 1, pl.ds(z0, RPT)], br1)
    lane = lax.iota(jnp.int32, LANES)
    perm = lane ^ HALF
    tail = jnp.where(lane == HALF, 1.0, 0.0)
    lo = lane < HALF

    def comb(i, carry):
        srow = bs0[i] + bs1[i]
        spl = jnp.full((LANES,), i, dtype=jnp.int32)
        rrot = (plsc.load_gather(br0, [spl, perm])
                + plsc.load_gather(br1, [spl, perm]))
        aggbuf[i] = jnp.where(lo, srow + rrot, tail)
        return carry

    lax.fori_loop(0, RPT, comb, 0)
    pltpu.sync_copy(aggbuf, agg_sh.at[pl.ds(z0, RPT)])
    plsc.subcore_barrier()

    def group(g, carry):
        descs = [
            pltpu.async_copy(agg_sh.at[idxo_v.at[g * GRP + bq]],
                             gbuf.at[pl.ds(bq * BLK, BLK)], sem_g)
            for bq in range(GRP)
        ]
        for d in descs:
            d.wait()
        descs = [
            pltpu.async_copy(gbuf.at[pl.ds(bq * BLK, BLK)],
                             sums_sh.at[idxi_v.at[g * GRP + bq]],
                             sem_s, add=True)
            for bq in range(GRP)
        ]
        for d in descs:
            d.wait()
        return carry

    lax.fori_loop(0, NGRP_B, group, 0)
    plsc.subcore_barrier()
    pltpu.sync_copy(sums_sh.at[pl.ds(z0, RPT)], tmp := bs0)
    pltpu.sync_copy(tmp, out_hbm.at[cid, pl.ds(z0, RPT)])


ROWS_BLK = 400


def _dense_body(x_ref, w_ref, b_ref, out_ref):
    out_ref[...] = jnp.dot(
        x_ref[...], w_ref[HALF:, :],
        preferred_element_type=jnp.float32) + b_ref[...]


def _finish_body(p1_ref, s0_ref, s1_ref, w_ref, out_ref):
    sums = s0_ref[0] + s1_ref[0]
    cnt = jnp.maximum(sums[:, HALF:HALF + 1], 1.0)
    navg = sums[:, :HALF] / cnt
    out_ref[...] = p1_ref[...] + jnp.dot(
        navg, w_ref[:HALF, :], preferred_element_type=jnp.float32)


def kernel(x, edge_attr, W, b, edge_index):
    senders = edge_index[0]
    receivers = edge_index[1]
    sidx = senders.reshape(CT, 128)
    ridx = receivers.reshape(CT, 128)
    ea4 = edge_attr.T.reshape(2, HALF, CT, 128).transpose(0, 2, 1, 3)
    zeros_np = jnp.zeros((NP, ROW_W), dtype=jnp.float32)

    grid = N_NODES // ROWS_BLK
    part1 = pl.pallas_call(
        _dense_body,
        grid=(grid,),
        in_specs=[
            pl.BlockSpec((ROWS_BLK, D_FEAT), lambda i: (i, 0)),
            pl.BlockSpec((D_FEAT + HALF, D_FEAT), lambda i: (0, 0)),
            pl.BlockSpec((1, D_FEAT), lambda i: (0, 0)),
        ],
        out_specs=pl.BlockSpec((ROWS_BLK, D_FEAT), lambda i: (i, 0)),
        out_shape=jax.ShapeDtypeStruct((N_NODES, D_FEAT), jnp.float32),
    )(x, W, b.reshape(1, D_FEAT))

    acc = _scatter_edges(ea4, sidx, ridx, zeros_np)

    idx_in = jnp.concatenate([senders, receivers]).reshape(NBLK_B, BLK)
    idx_out = jnp.concatenate([receivers, senders]).reshape(NBLK_B, BLK)
    sums_pair = _gather_scatter_add(acc, idx_out, idx_in, zeros_np)

    out = pl.pallas_call(
        _finish_body,
        grid=(grid,),
        in_specs=[
            pl.BlockSpec((ROWS_BLK, D_FEAT), lambda i: (i, 0)),
            pl.BlockSpec((1, ROWS_BLK, ROW_W), lambda i: (0, i, 0)),
            pl.BlockSpec((1, ROWS_BLK, ROW_W), lambda i: (1, i, 0)),
            pl.BlockSpec((D_FEAT + HALF, D_FEAT), lambda i: (0, 0)),
        ],
        out_specs=pl.BlockSpec((ROWS_BLK, D_FEAT), lambda i: (i, 0)),
        out_shape=jax.ShapeDtypeStruct((N_NODES, D_FEAT), jnp.float32),
    )(part1, sums_pair, sums_pair, W)
    return out


# speedup vs baseline: 1.6283x; 1.0407x over previous
import functools

import jax
import jax.numpy as jnp
from jax import lax
from jax.experimental import pallas as pl
from jax.experimental.pallas import tpu as pltpu
from jax.experimental.pallas import tpu_sc as plsc

N_NODES = 10000
N_EDGES = 160000
TWO_E = 2 * N_EDGES
D_FEAT = 256
HALF = 8
ROW_W = 16
LANES = 16

NUM_CORES = 2
NUM_SUBCORES = 16
NUM_WORKERS = NUM_CORES * NUM_SUBCORES
BLK = 125
GRP = 8

CT = N_EDGES // 128
CT_BASE = CT // NUM_WORKERS
CT_MAX = CT_BASE + 1

BPW_B = TWO_E // (NUM_WORKERS * BLK)
NBLK_B = TWO_E // BLK
NGRP_B = BPW_B // GRP

NP = N_NODES
RPT = NP // NUM_SUBCORES


def _worker_id():
    return lax.axis_index("c") * NUM_SUBCORES + lax.axis_index("s")


_MESH = plsc.VectorSubcoreMesh(core_axis_name="c", subcore_axis_name="s")
_SC_PARAMS = pltpu.CompilerParams(use_tc_tiling_on_sc=False,
                                  needs_layout_passes=False)


@functools.partial(
    pl.kernel,
    out_type=jax.ShapeDtypeStruct((NUM_CORES, 2, NP, ROW_W), jnp.float32),
    mesh=_MESH,
    scratch_types=[
        pltpu.VMEM((CT_MAX, 128), jnp.int32),
        pltpu.VMEM((CT_MAX, 128), jnp.int32),
        pltpu.VMEM((2, CT_MAX, HALF, 128), jnp.float32),
        pltpu.VMEM((4 * 128, ROW_W), jnp.float32),
        pltpu.VMEM((RPT, ROW_W), jnp.float32),
        pltpu.VMEM_SHARED((NP, ROW_W), jnp.float32),
        pltpu.VMEM_SHARED((NP, ROW_W), jnp.float32),
        pltpu.SemaphoreType.DMA,
        pltpu.SemaphoreType.DMA,
    ],
    compiler_params=_SC_PARAMS,
)
def _scatter_edges(ea4_hbm, sidx_hbm, ridx_hbm, zeros_hbm, out_hbm,
                   sidx_v, ridx_v, fmbuf, rowbuf, tmp,
                   accs_sh, accr_sh, sem_ld, sem_st):
    cid = lax.axis_index("c")
    sid = lax.axis_index("s")
    w = _worker_id()
    z0 = sid * RPT
    nct = CT_BASE + jnp.where(w < CT % NUM_WORKERS, 1, 0)
    base = CT_BASE * w + jnp.minimum(w, CT % NUM_WORKERS)
    loads = [
        pltpu.async_copy(ea4_hbm.at[0, pl.ds(base, CT_BASE)],
                         fmbuf.at[0, pl.ds(0, CT_BASE)], sem_ld),
        pltpu.async_copy(ea4_hbm.at[1, pl.ds(base, CT_BASE)],
                         fmbuf.at[1, pl.ds(0, CT_BASE)], sem_ld),
        pltpu.async_copy(sidx_hbm.at[pl.ds(base, CT_BASE)],
                         sidx_v.at[pl.ds(0, CT_BASE)], sem_ld),
        pltpu.async_copy(ridx_hbm.at[pl.ds(base, CT_BASE)],
                         ridx_v.at[pl.ds(0, CT_BASE)], sem_ld),
    ]
    pltpu.sync_copy(zeros_hbm.at[pl.ds(z0, RPT)], accs_sh.at[pl.ds(z0, RPT)])
    pltpu.sync_copy(zeros_hbm.at[pl.ds(z0, RPT)], accr_sh.at[pl.ds(z0, RPT)])

    @pl.when(w < CT % NUM_WORKERS)
    def _():
        pltpu.sync_copy(ea4_hbm.at[0, pl.ds(base + CT_BASE, 1)],
                        fmbuf.at[0, pl.ds(CT_BASE, 1)])
        pltpu.sync_copy(ea4_hbm.at[1, pl.ds(base + CT_BASE, 1)],
                        fmbuf.at[1, pl.ds(CT_BASE, 1)])
        pltpu.sync_copy(sidx_hbm.at[pl.ds(base + CT_BASE, 1)],
                        sidx_v.at[pl.ds(CT_BASE, 1)])
        pltpu.sync_copy(ridx_hbm.at[pl.ds(base + CT_BASE, 1)],
                        ridx_v.at[pl.ds(CT_BASE, 1)])

    for d in loads:
        d.wait()
    plsc.subcore_barrier()

    lane = lax.iota(jnp.int32, LANES)
    tvec = jnp.where(lane < HALF, 0, 1)
    rvec = lane & (HALF - 1)

    def coltile(j, carry):
        off = (j & 3) * 128
        @pl.when(j >= 4)
        def _():
            pltpu.make_async_copy(zeros_hbm.at[pl.ds(0, 128)],
                                  rowbuf.at[pl.ds(off, 128)], sem_st).wait()
            pltpu.make_async_copy(zeros_hbm.at[pl.ds(0, 128)],
                                  rowbuf.at[pl.ds(off, 128)], sem_st).wait()

        jvec = jnp.full((LANES,), j, dtype=jnp.int32)

        def transpose16(eh, c2):
            ebase = jnp.full((LANES,), eh * LANES, dtype=jnp.int32)
            vals = [plsc.load_gather(fmbuf, [tvec, jvec, rvec, ebase + el])
                    for el in range(LANES)]
            for el in range(LANES):
                rowbuf[off + eh * LANES + el] = vals[el]
            return c2

        lax.fori_loop(0, 128 // LANES, transpose16, 0)
        src = rowbuf.at[pl.ds(off, 128)]
        pltpu.async_copy(src, accs_sh.at[sidx_v.at[j]], sem_st, add=True)
        pltpu.async_copy(src, accr_sh.at[ridx_v.at[j]], sem_st, add=True)
        return carry

    lax.fori_loop(0, nct, coltile, 0)
    for _ in range(8):
        pltpu.make_async_copy(zeros_hbm.at[pl.ds(0, 128)],
                              rowbuf.at[pl.ds(0, 128)], sem_st).wait()
    plsc.subcore_barrier()
    pltpu.sync_copy(accs_sh.at[pl.ds(z0, RPT)], tmp)
    pltpu.sync_copy(tmp, out_hbm.at[cid, 0, pl.ds(z0, RPT)])
    pltpu.sync_copy(accr_sh.at[pl.ds(z0, RPT)], tmp)
    pltpu.sync_copy(tmp, out_hbm.at[cid, 1, pl.ds(z0, RPT)])


@functools.partial(
    pl.kernel,
    out_type=jax.ShapeDtypeStruct((NUM_CORES, NP, ROW_W), jnp.float32),
    mesh=_MESH,
    scratch_types=[
        pltpu.VMEM((BPW_B, BLK), jnp.int32),
        pltpu.VMEM((BPW_B, BLK), jnp.int32),
        pltpu.VMEM((2 * GRP * BLK, ROW_W), jnp.float32),
        pltpu.VMEM((RPT, ROW_W), jnp.float32),
        pltpu.VMEM((RPT, ROW_W), jnp.float32),
        pltpu.VMEM((RPT, ROW_W), jnp.float32),
        pltpu.VMEM((RPT, ROW_W), jnp.float32),
        pltpu.VMEM((RPT, ROW_W), jnp.float32),
        pltpu.VMEM_SHARED((NP, ROW_W), jnp.float32),
        pltpu.VMEM_SHARED((NP, ROW_W), jnp.float32),
        pltpu.SemaphoreType.DMA,
        pltpu.SemaphoreType.DMA,
    ],
    compiler_params=_SC_PARAMS,
)
def _gather_scatter_add(acc_hbm, idxo_hbm, idxi_hbm, zeros_hbm, out_hbm,
                        idxo_v, idxi_v, gbuf, bs0, bs1, br0, br1, aggbuf,
                        agg_sh, sums_sh, sem_g, sem_s):
    cid = lax.axis_index("c")
    sid = lax.axis_index("s")
    w = _worker_id()
    z0 = sid * RPT
    pltpu.sync_copy(zeros_hbm.at[pl.ds(z0, RPT)], sums_sh.at[pl.ds(z0, RPT)])
    pltpu.sync_copy(idxo_hbm.at[pl.ds(w * BPW_B, BPW_B)], idxo_v)
    pltpu.sync_copy(idxi_hbm.at[pl.ds(w * BPW_B, BPW_B)], idxi_v)
    pltpu.sync_copy(acc_hbm.at[0, 0, pl.ds(z0, RPT)], bs0)
    pltpu.sync_copy(acc_hbm.at[1, 0, pl.ds(z0, RPT)], bs1)
    pltpu.sync_copy(acc_hbm.at[0, 1, pl.ds(z0, RPT)], br0)
    pltpu.sync_copy(acc_hbm.at[1, 1, pl.ds(z0, RPT)], br1)
    lane = lax.iota(jnp.int32, LANES)
    perm = lane ^ HALF
    tail = jnp.where(lane == HALF, 1.0, 0.0)
    lo = lane < HALF

    def comb(i, carry):
        srow = bs0[i] + bs1[i]
        spl = jnp.full((LANES,), i, dtype=jnp.int32)
        rrot = (plsc.load_gather(br0, [spl, perm])
                + plsc.load_gather(br1, [spl, perm]))
        aggbuf[i] = jnp.where(lo, srow + rrot, tail)
        return carry

    lax.fori_loop(0, RPT, comb, 0)
    pltpu.sync_copy(aggbuf, agg_sh.at[pl.ds(z0, RPT)])
    plsc.subcore_barrier()

    for bq in range(GRP):
        pltpu.async_copy(agg_sh.at[idxo_v.at[bq]],
                         gbuf.at[pl.ds(bq * BLK, BLK)], sem_g)

    def group(g, carry):
        off = (g & 1) * (GRP * BLK)
        for _ in range(GRP):
            pltpu.make_async_copy(zeros_hbm.at[pl.ds(0, BLK)],
                                  gbuf.at[pl.ds(0, BLK)], sem_g).wait()

        @pl.when(g >= 1)
        def _():
            for _ in range(GRP):
                pltpu.make_async_copy(zeros_hbm.at[pl.ds(0, BLK)],
                                      gbuf.at[pl.ds(0, BLK)], sem_s).wait()

        @pl.when(g + 1 < NGRP_B)
        def _():
            off2 = (GRP * BLK) - off
            for bq in range(GRP):
                pltpu.async_copy(agg_sh.at[idxo_v.at[(g + 1) * GRP + bq]],
                                 gbuf.at[pl.ds(off2 + bq * BLK, BLK)], sem_g)

        for bq in range(GRP):
            pltpu.async_copy(gbuf.at[pl.ds(off + bq * BLK, BLK)],
                             sums_sh.at[idxi_v.at[g * GRP + bq]],
                             sem_s, add=True)
        return carry

    lax.fori_loop(0, NGRP_B, group, 0)
    for _ in range(GRP):
        pltpu.make_async_copy(zeros_hbm.at[pl.ds(0, BLK)],
                              gbuf.at[pl.ds(0, BLK)], sem_s).wait()
    plsc.subcore_barrier()
    pltpu.sync_copy(sums_sh.at[pl.ds(z0, RPT)], tmp := bs0)
    pltpu.sync_copy(tmp, out_hbm.at[cid, pl.ds(z0, RPT)])


ROWS_BLK = 400


def _dense_body(x_ref, w_ref, b_ref, out_ref):
    out_ref[...] = jnp.dot(
        x_ref[...], w_ref[HALF:, :],
        preferred_element_type=jnp.float32) + b_ref[...]


def _finish_body(p1_ref, s0_ref, s1_ref, w_ref, out_ref):
    sums = s0_ref[0] + s1_ref[0]
    cnt = jnp.maximum(sums[:, HALF:HALF + 1], 1.0)
    navg = sums[:, :HALF] / cnt
    out_ref[...] = p1_ref[...] + jnp.dot(
        navg, w_ref[:HALF, :], preferred_element_type=jnp.float32)


def kernel(x, edge_attr, W, b, edge_index):
    senders = edge_index[0]
    receivers = edge_index[1]
    sidx = senders.reshape(CT, 128)
    ridx = receivers.reshape(CT, 128)
    ea4 = edge_attr.T.reshape(2, HALF, CT, 128).transpose(0, 2, 1, 3)
    zeros_np = jnp.zeros((NP, ROW_W), dtype=jnp.float32)

    grid = N_NODES // ROWS_BLK
    part1 = pl.pallas_call(
        _dense_body,
        grid=(grid,),
        in_specs=[
            pl.BlockSpec((ROWS_BLK, D_FEAT), lambda i: (i, 0)),
            pl.BlockSpec((D_FEAT + HALF, D_FEAT), lambda i: (0, 0)),
            pl.BlockSpec((1, D_FEAT), lambda i: (0, 0)),
        ],
        out_specs=pl.BlockSpec((ROWS_BLK, D_FEAT), lambda i: (i, 0)),
        out_shape=jax.ShapeDtypeStruct((N_NODES, D_FEAT), jnp.float32),
    )(x, W, b.reshape(1, D_FEAT))

    acc = _scatter_edges(ea4, sidx, ridx, zeros_np)

    idx_in = jnp.concatenate([senders, receivers]).reshape(NBLK_B, BLK)
    idx_out = jnp.concatenate([receivers, senders]).reshape(NBLK_B, BLK)
    sums_pair = _gather_scatter_add(acc, idx_out, idx_in, zeros_np)

    out = pl.pallas_call(
        _finish_body,
        grid=(grid,),
        in_specs=[
            pl.BlockSpec((ROWS_BLK, D_FEAT), lambda i: (i, 0)),
            pl.BlockSpec((1, ROWS_BLK, ROW_W), lambda i: (0, i, 0)),
            pl.BlockSpec((1, ROWS_BLK, ROW_W), lambda i: (1, i, 0)),
            pl.BlockSpec((D_FEAT + HALF, D_FEAT), lambda i: (0, 0)),
        ],
        out_specs=pl.BlockSpec((ROWS_BLK, D_FEAT), lambda i: (i, 0)),
        out_shape=jax.ShapeDtypeStruct((N_NODES, D_FEAT), jnp.float32),
    )(part1, sums_pair, sums_pair, W)
    return out


# speedup vs baseline: 1.8257x; 1.1212x over previous
import functools

import jax
import jax.numpy as jnp
from jax import lax
from jax.experimental import pallas as pl
from jax.experimental.pallas import tpu as pltpu
from jax.experimental.pallas import tpu_sc as plsc

N_NODES = 10000
N_EDGES = 160000
TWO_E = 2 * N_EDGES
D_FEAT = 256
HALF = 8
ROW_W = 16
LANES = 16

NUM_CORES = 2
NUM_SUBCORES = 16
NUM_WORKERS = NUM_CORES * NUM_SUBCORES
BLK = 125
GRP = 8

CT = N_EDGES // 128
CT_BASE = CT // NUM_WORKERS
CT_MAX = CT_BASE + 1

BPW_B = TWO_E // (NUM_WORKERS * BLK)
NBLK_B = TWO_E // BLK
NGRP_B = BPW_B // GRP

NP = N_NODES
RPT = NP // NUM_SUBCORES


def _worker_id():
    return lax.axis_index("c") * NUM_SUBCORES + lax.axis_index("s")


_MESH = plsc.VectorSubcoreMesh(core_axis_name="c", subcore_axis_name="s")
_SC_PARAMS = pltpu.CompilerParams(use_tc_tiling_on_sc=False,
                                  needs_layout_passes=False)


@functools.partial(
    pl.kernel,
    out_type=jax.ShapeDtypeStruct((NUM_CORES, 2, NP, ROW_W), jnp.float32),
    mesh=_MESH,
    scratch_types=[
        pltpu.VMEM((CT_MAX, 128), jnp.int32),
        pltpu.VMEM((CT_MAX, 128), jnp.int32),
        pltpu.VMEM((2, CT_MAX, HALF, 128), jnp.float32),
        pltpu.VMEM((2 * 128, ROW_W), jnp.float32),
        pltpu.VMEM((RPT, ROW_W), jnp.float32),
        pltpu.VMEM_SHARED((NP, ROW_W), jnp.float32),
        pltpu.VMEM_SHARED((NP, ROW_W), jnp.float32),
        pltpu.SemaphoreType.DMA,
        pltpu.SemaphoreType.DMA,
    ],
    compiler_params=_SC_PARAMS,
)
def _scatter_edges(ea4_hbm, sidx_hbm, ridx_hbm, zeros_hbm, out_hbm,
                   sidx_v, ridx_v, fmbuf, rowbuf, tmp,
                   accs_sh, accr_sh, sem_ld, sem_st):
    cid = lax.axis_index("c")
    sid = lax.axis_index("s")
    w = _worker_id()
    z0 = sid * RPT
    nct = CT_BASE + jnp.where(w < CT % NUM_WORKERS, 1, 0)
    base = CT_BASE * w + jnp.minimum(w, CT % NUM_WORKERS)
    loads = [
        pltpu.async_copy(ea4_hbm.at[0, pl.ds(base, CT_BASE)],
                         fmbuf.at[0, pl.ds(0, CT_BASE)], sem_ld),
        pltpu.async_copy(ea4_hbm.at[1, pl.ds(base, CT_BASE)],
                         fmbuf.at[1, pl.ds(0, CT_BASE)], sem_ld),
        pltpu.async_copy(sidx_hbm.at[pl.ds(base, CT_BASE)],
                         sidx_v.at[pl.ds(0, CT_BASE)], sem_ld),
        pltpu.async_copy(ridx_hbm.at[pl.ds(base, CT_BASE)],
                         ridx_v.at[pl.ds(0, CT_BASE)], sem_ld),
    ]
    pltpu.sync_copy(zeros_hbm.at[pl.ds(z0, RPT)], accs_sh.at[pl.ds(z0, RPT)])
    pltpu.sync_copy(zeros_hbm.at[pl.ds(z0, RPT)], accr_sh.at[pl.ds(z0, RPT)])

    @pl.when(w < CT % NUM_WORKERS)
    def _():
        pltpu.sync_copy(ea4_hbm.at[0, pl.ds(base + CT_BASE, 1)],
                        fmbuf.at[0, pl.ds(CT_BASE, 1)])
        pltpu.sync_copy(ea4_hbm.at[1, pl.ds(base + CT_BASE, 1)],
                        fmbuf.at[1, pl.ds(CT_BASE, 1)])
        pltpu.sync_copy(sidx_hbm.at[pl.ds(base + CT_BASE, 1)],
                        sidx_v.at[pl.ds(CT_BASE, 1)])
        pltpu.sync_copy(ridx_hbm.at[pl.ds(base + CT_BASE, 1)],
                        ridx_v.at[pl.ds(CT_BASE, 1)])

    for d in loads:
        d.wait()
    plsc.subcore_barrier()

    lane = lax.iota(jnp.int32, LANES)
    evecs = [lane + eh * LANES for eh in range(8)]

    def coltile(j, carry):
        off = (j & 1) * 128
        @pl.when(j >= 2)
        def _():
            pltpu.make_async_copy(zeros_hbm.at[pl.ds(0, 128)],
                                  rowbuf.at[pl.ds(off, 128)], sem_st).wait()
            pltpu.make_async_copy(zeros_hbm.at[pl.ds(0, 128)],
                                  rowbuf.at[pl.ds(off, 128)], sem_st).wait()

        jvec = jnp.full((LANES,), j, dtype=jnp.int32)
        offv = jnp.full((LANES,), off, dtype=jnp.int32)
        def diag(d, c2):
            kl = (lane + d) & (LANES - 1)
            tv = kl >> 3
            rv = kl & (HALF - 1)
            for eh in range(8):
                val = plsc.load_gather(fmbuf, [tv, jvec, rv, evecs[eh]])
                plsc.store_scatter(rowbuf, [offv + evecs[eh], kl], val)
            return c2

        lax.fori_loop(0, LANES, diag, 0)
        src = rowbuf.at[pl.ds(off, 128)]
        pltpu.async_copy(src, accs_sh.at[sidx_v.at[j]], sem_st, add=True)
        pltpu.async_copy(src, accr_sh.at[ridx_v.at[j]], sem_st, add=True)
        return carry

    lax.fori_loop(0, nct, coltile, 0)
    for _ in range(4):
        pltpu.make_async_copy(zeros_hbm.at[pl.ds(0, 128)],
                              rowbuf.at[pl.ds(0, 128)], sem_st).wait()
    plsc.subcore_barrier()
    pltpu.sync_copy(accs_sh.at[pl.ds(z0, RPT)], tmp)
    pltpu.sync_copy(tmp, out_hbm.at[cid, 0, pl.ds(z0, RPT)])
    pltpu.sync_copy(accr_sh.at[pl.ds(z0, RPT)], tmp)
    pltpu.sync_copy(tmp, out_hbm.at[cid, 1, pl.ds(z0, RPT)])


@functools.partial(
    pl.kernel,
    out_type=jax.ShapeDtypeStruct((NUM_CORES, NP, ROW_W), jnp.float32),
    mesh=_MESH,
    scratch_types=[
        pltpu.VMEM((BPW_B, BLK), jnp.int32),
        pltpu.VMEM((BPW_B, BLK), jnp.int32),
        pltpu.VMEM((2 * GRP * BLK, ROW_W), jnp.float32),
        pltpu.VMEM((RPT, ROW_W), jnp.float32),
        pltpu.VMEM((RPT, ROW_W), jnp.float32),
        pltpu.VMEM((RPT, ROW_W), jnp.float32),
        pltpu.VMEM((RPT, ROW_W), jnp.float32),
        pltpu.VMEM((RPT, ROW_W), jnp.float32),
        pltpu.VMEM_SHARED((NP, ROW_W), jnp.float32),
        pltpu.VMEM_SHARED((NP, ROW_W), jnp.float32),
        pltpu.SemaphoreType.DMA,
        pltpu.SemaphoreType.DMA,
    ],
    compiler_params=_SC_PARAMS,
)
def _gather_scatter_add(acc_hbm, idxo_hbm, idxi_hbm, zeros_hbm, out_hbm,
                        idxo_v, idxi_v, gbuf, bs0, bs1, br0, br1, aggbuf,
                        agg_sh, sums_sh, sem_g, sem_s):
    cid = lax.axis_index("c")
    sid = lax.axis_index("s")
    w = _worker_id()
    z0 = sid * RPT
    pltpu.sync_copy(zeros_hbm.at[pl.ds(z0, RPT)], sums_sh.at[pl.ds(z0, RPT)])
    pltpu.sync_copy(idxo_hbm.at[pl.ds(w * BPW_B, BPW_B)], idxo_v)
    pltpu.sync_copy(idxi_hbm.at[pl.ds(w * BPW_B, BPW_B)], idxi_v)
    pltpu.sync_copy(acc_hbm.at[0, 0, pl.ds(z0, RPT)], bs0)
    pltpu.sync_copy(acc_hbm.at[1, 0, pl.ds(z0, RPT)], bs1)
    pltpu.sync_copy(acc_hbm.at[0, 1, pl.ds(z0, RPT)], br0)
    pltpu.sync_copy(acc_hbm.at[1, 1, pl.ds(z0, RPT)], br1)
    lane = lax.iota(jnp.int32, LANES)
    perm = lane ^ HALF
    tail = jnp.where(lane == HALF, 1.0, 0.0)
    lo = lane < HALF

    def comb(i, carry):
        srow = bs0[i] + bs1[i]
        spl = jnp.full((LANES,), i, dtype=jnp.int32)
        rrot = (plsc.load_gather(br0, [spl, perm])
                + plsc.load_gather(br1, [spl, perm]))
        aggbuf[i] = jnp.where(lo, srow + rrot, tail)
        return carry

    lax.fori_loop(0, RPT, comb, 0)
    pltpu.sync_copy(aggbuf, agg_sh.at[pl.ds(z0, RPT)])
    plsc.subcore_barrier()

    for bq in range(GRP):
        pltpu.async_copy(agg_sh.at[idxo_v.at[bq]],
                         gbuf.at[pl.ds(bq * BLK, BLK)], sem_g)

    def group(g, carry):
        off = (g & 1) * (GRP * BLK)
        for _ in range(GRP):
            pltpu.make_async_copy(zeros_hbm.at[pl.ds(0, BLK)],
                                  gbuf.at[pl.ds(0, BLK)], sem_g).wait()

        @pl.when(g >= 1)
        def _():
            for _ in range(GRP):
                pltpu.make_async_copy(zeros_hbm.at[pl.ds(0, BLK)],
                                      gbuf.at[pl.ds(0, BLK)], sem_s).wait()

        @pl.when(g + 1 < NGRP_B)
        def _():
            off2 = (GRP * BLK) - off
            for bq in range(GRP):
                pltpu.async_copy(agg_sh.at[idxo_v.at[(g + 1) * GRP + bq]],
                                 gbuf.at[pl.ds(off2 + bq * BLK, BLK)], sem_g)

        for bq in range(GRP):
            pltpu.async_copy(gbuf.at[pl.ds(off + bq * BLK, BLK)],
                             sums_sh.at[idxi_v.at[g * GRP + bq]],
                             sem_s, add=True)
        return carry

    lax.fori_loop(0, NGRP_B, group, 0)
    for _ in range(GRP):
        pltpu.make_async_copy(zeros_hbm.at[pl.ds(0, BLK)],
                              gbuf.at[pl.ds(0, BLK)], sem_s).wait()
    plsc.subcore_barrier()
    pltpu.sync_copy(sums_sh.at[pl.ds(z0, RPT)], tmp := bs0)
    pltpu.sync_copy(tmp, out_hbm.at[cid, pl.ds(z0, RPT)])


ROWS_BLK = 400


def _dense_body(x_ref, w_ref, b_ref, out_ref):
    out_ref[...] = jnp.dot(
        x_ref[...], w_ref[HALF:, :],
        preferred_element_type=jnp.float32) + b_ref[...]


def _finish_body(p1_ref, s0_ref, s1_ref, w_ref, out_ref):
    sums = s0_ref[0] + s1_ref[0]
    cnt = jnp.maximum(sums[:, HALF:HALF + 1], 1.0)
    navg = sums[:, :HALF] / cnt
    out_ref[...] = p1_ref[...] + jnp.dot(
        navg, w_ref[:HALF, :], preferred_element_type=jnp.float32)


def kernel(x, edge_attr, W, b, edge_index):
    senders = edge_index[0]
    receivers = edge_index[1]
    sidx = senders.reshape(CT, 128)
    ridx = receivers.reshape(CT, 128)
    ea4 = edge_attr.T.reshape(2, HALF, CT, 128).transpose(0, 2, 1, 3)
    zeros_np = jnp.zeros((NP, ROW_W), dtype=jnp.float32)

    grid = N_NODES // ROWS_BLK
    part1 = pl.pallas_call(
        _dense_body,
        grid=(grid,),
        in_specs=[
            pl.BlockSpec((ROWS_BLK, D_FEAT), lambda i: (i, 0)),
            pl.BlockSpec((D_FEAT + HALF, D_FEAT), lambda i: (0, 0)),
            pl.BlockSpec((1, D_FEAT), lambda i: (0, 0)),
        ],
        out_specs=pl.BlockSpec((ROWS_BLK, D_FEAT), lambda i: (i, 0)),
        out_shape=jax.ShapeDtypeStruct((N_NODES, D_FEAT), jnp.float32),
    )(x, W, b.reshape(1, D_FEAT))

    acc = _scatter_edges(ea4, sidx, ridx, zeros_np)

    idx_in = jnp.concatenate([senders, receivers]).reshape(NBLK_B, BLK)
    idx_out = jnp.concatenate([receivers, senders]).reshape(NBLK_B, BLK)
    sums_pair = _gather_scatter_add(acc, idx_out, idx_in, zeros_np)

    out = pl.pallas_call(
        _finish_body,
        grid=(grid,),
        in_specs=[
            pl.BlockSpec((ROWS_BLK, D_FEAT), lambda i: (i, 0)),
            pl.BlockSpec((1, ROWS_BLK, ROW_W), lambda i: (0, i, 0)),
            pl.BlockSpec((1, ROWS_BLK, ROW_W), lambda i: (1, i, 0)),
            pl.BlockSpec((D_FEAT + HALF, D_FEAT), lambda i: (0, 0)),
        ],
        out_specs=pl.BlockSpec((ROWS_BLK, D_FEAT), lambda i: (i, 0)),
        out_shape=jax.ShapeDtypeStruct((N_NODES, D_FEAT), jnp.float32),
    )(part1, sums_pair, sums_pair, W)
    return out


# speedup vs baseline: 1.9434x; 1.0645x over previous
import functools

import jax
import jax.numpy as jnp
from jax import lax
from jax.experimental import pallas as pl
from jax.experimental.pallas import tpu as pltpu
from jax.experimental.pallas import tpu_sc as plsc

N_NODES = 10000
N_EDGES = 160000
TWO_E = 2 * N_EDGES
D_FEAT = 256
HALF = 8
ROW_W = 16
LANES = 16

NUM_CORES = 2
NUM_SUBCORES = 16
NUM_WORKERS = NUM_CORES * NUM_SUBCORES
BLK = 125
GRP = 8

CT = N_EDGES // 128
CT_BASE = CT // NUM_WORKERS
CT_MAX = CT_BASE + 1

BPW_B = TWO_E // (NUM_WORKERS * BLK)
NBLK_B = TWO_E // BLK
NGRP_B = BPW_B // GRP

NP = N_NODES
RPT = NP // NUM_SUBCORES


def _worker_id():
    return lax.axis_index("c") * NUM_SUBCORES + lax.axis_index("s")


_MESH = plsc.VectorSubcoreMesh(core_axis_name="c", subcore_axis_name="s")
_SC_PARAMS = pltpu.CompilerParams(use_tc_tiling_on_sc=False,
                                  needs_layout_passes=False)


@functools.partial(
    pl.kernel,
    out_type=jax.ShapeDtypeStruct((NUM_CORES, 2, NP, ROW_W), jnp.float32),
    mesh=_MESH,
    scratch_types=[
        pltpu.VMEM((CT_MAX, 128), jnp.int32),
        pltpu.VMEM((CT_MAX, 128), jnp.int32),
        pltpu.VMEM((2, CT_MAX, HALF, 128), jnp.float32),
        pltpu.VMEM((2 * 128, ROW_W), jnp.float32),
        pltpu.VMEM((RPT, ROW_W), jnp.float32),
        pltpu.VMEM_SHARED((NP, ROW_W), jnp.float32),
        pltpu.VMEM_SHARED((NP, ROW_W), jnp.float32),
        pltpu.SemaphoreType.DMA,
        pltpu.SemaphoreType.DMA,
    ],
    compiler_params=_SC_PARAMS,
)
def _scatter_edges(ea4_hbm, sidx_hbm, ridx_hbm, zeros_hbm, out_hbm,
                   sidx_v, ridx_v, fmbuf, rowbuf, tmp,
                   accs_sh, accr_sh, sem_ld, sem_st):
    cid = lax.axis_index("c")
    sid = lax.axis_index("s")
    w = _worker_id()
    z0 = sid * RPT
    nct = CT_BASE + jnp.where(w < CT % NUM_WORKERS, 1, 0)
    base = CT_BASE * w + jnp.minimum(w, CT % NUM_WORKERS)
    loads = [
        pltpu.async_copy(ea4_hbm.at[0, pl.ds(base, CT_BASE)],
                         fmbuf.at[0, pl.ds(0, CT_BASE)], sem_ld),
        pltpu.async_copy(ea4_hbm.at[1, pl.ds(base, CT_BASE)],
                         fmbuf.at[1, pl.ds(0, CT_BASE)], sem_ld),
        pltpu.async_copy(sidx_hbm.at[pl.ds(base, CT_BASE)],
                         sidx_v.at[pl.ds(0, CT_BASE)], sem_ld),
        pltpu.async_copy(ridx_hbm.at[pl.ds(base, CT_BASE)],
                         ridx_v.at[pl.ds(0, CT_BASE)], sem_ld),
    ]
    pltpu.sync_copy(zeros_hbm.at[pl.ds(z0, RPT)], accs_sh.at[pl.ds(z0, RPT)])
    pltpu.sync_copy(zeros_hbm.at[pl.ds(z0, RPT)], accr_sh.at[pl.ds(z0, RPT)])

    @pl.when(w < CT % NUM_WORKERS)
    def _():
        pltpu.sync_copy(ea4_hbm.at[0, pl.ds(base + CT_BASE, 1)],
                        fmbuf.at[0, pl.ds(CT_BASE, 1)])
        pltpu.sync_copy(ea4_hbm.at[1, pl.ds(base + CT_BASE, 1)],
                        fmbuf.at[1, pl.ds(CT_BASE, 1)])
        pltpu.sync_copy(sidx_hbm.at[pl.ds(base + CT_BASE, 1)],
                        sidx_v.at[pl.ds(CT_BASE, 1)])
        pltpu.sync_copy(ridx_hbm.at[pl.ds(base + CT_BASE, 1)],
                        ridx_v.at[pl.ds(CT_BASE, 1)])

    for d in loads:
        d.wait()
    plsc.subcore_barrier()

    lane = lax.iota(jnp.int32, LANES)
    evecs = [lane + eh * LANES for eh in range(8)]

    def coltile(j, carry):
        off = (j & 1) * 128
        @pl.when(j >= 2)
        def _():
            pltpu.make_async_copy(zeros_hbm.at[pl.ds(0, 128)],
                                  rowbuf.at[pl.ds(off, 128)], sem_st).wait()
            pltpu.make_async_copy(zeros_hbm.at[pl.ds(0, 128)],
                                  rowbuf.at[pl.ds(off, 128)], sem_st).wait()

        jvec = jnp.full((LANES,), j, dtype=jnp.int32)
        offv = jnp.full((LANES,), off, dtype=jnp.int32)
        def diag(d, c2):
            kl = (lane + d) & (LANES - 1)
            tv = kl >> 3
            rv = kl & (HALF - 1)
            for eh in range(8):
                val = plsc.load_gather(fmbuf, [tv, jvec, rv, evecs[eh]])
                plsc.store_scatter(rowbuf, [offv + evecs[eh], kl], val)
            return c2

        lax.fori_loop(0, LANES, diag, 0)
        src = rowbuf.at[pl.ds(off, 128)]
        pltpu.async_copy(src, accs_sh.at[sidx_v.at[j]], sem_st, add=True)
        pltpu.async_copy(src, accr_sh.at[ridx_v.at[j]], sem_st, add=True)
        return carry

    lax.fori_loop(0, nct, coltile, 0)
    for _ in range(4):
        pltpu.make_async_copy(zeros_hbm.at[pl.ds(0, 128)],
                              rowbuf.at[pl.ds(0, 128)], sem_st).wait()
    plsc.subcore_barrier()
    pltpu.sync_copy(accs_sh.at[pl.ds(z0, RPT)], tmp)
    pltpu.sync_copy(tmp, out_hbm.at[cid, 0, pl.ds(z0, RPT)])
    pltpu.sync_copy(accr_sh.at[pl.ds(z0, RPT)], tmp)
    pltpu.sync_copy(tmp, out_hbm.at[cid, 1, pl.ds(z0, RPT)])


@functools.partial(
    pl.kernel,
    out_type=jax.ShapeDtypeStruct((NUM_CORES, NP, ROW_W), jnp.float32),
    mesh=_MESH,
    scratch_types=[
        pltpu.VMEM((BPW_B, BLK), jnp.int32),
        pltpu.VMEM((BPW_B, BLK), jnp.int32),
        pltpu.VMEM((2 * GRP * BLK, ROW_W), jnp.float32),
        pltpu.VMEM((RPT, ROW_W), jnp.float32),
        pltpu.VMEM((RPT, ROW_W), jnp.float32),
        pltpu.VMEM((RPT, ROW_W), jnp.float32),
        pltpu.VMEM((RPT, ROW_W), jnp.float32),
        pltpu.VMEM((RPT, ROW_W), jnp.float32),
        pltpu.VMEM_SHARED((NP, ROW_W), jnp.float32),
        pltpu.VMEM_SHARED((NP, ROW_W), jnp.float32),
        pltpu.SemaphoreType.DMA,
        pltpu.SemaphoreType.DMA,
    ],
    compiler_params=_SC_PARAMS,
)
def _gather_scatter_add(acc_hbm, idxo_hbm, idxi_hbm, zeros_hbm, out_hbm,
                        idxo_v, idxi_v, gbuf, bs0, bs1, br0, br1, aggbuf,
                        agg_sh, sums_sh, sem_g, sem_s):
    cid = lax.axis_index("c")
    sid = lax.axis_index("s")
    w = _worker_id()
    z0 = sid * RPT
    pltpu.sync_copy(zeros_hbm.at[pl.ds(z0, RPT)], sums_sh.at[pl.ds(z0, RPT)])
    pltpu.sync_copy(idxo_hbm.at[pl.ds(w * BPW_B, BPW_B)], idxo_v)
    pltpu.sync_copy(idxi_hbm.at[pl.ds(w * BPW_B, BPW_B)], idxi_v)
    pltpu.sync_copy(acc_hbm.at[0, 0, pl.ds(z0, RPT)], bs0)
    pltpu.sync_copy(acc_hbm.at[1, 0, pl.ds(z0, RPT)], bs1)
    pltpu.sync_copy(acc_hbm.at[0, 1, pl.ds(z0, RPT)], br0)
    pltpu.sync_copy(acc_hbm.at[1, 1, pl.ds(z0, RPT)], br1)
    lane = lax.iota(jnp.int32, LANES)
    perm = lane ^ HALF
    tail = jnp.where(lane == HALF, 1.0, 0.0)
    lo = lane < HALF

    def comb(i, carry):
        srow = bs0[i] + bs1[i]
        spl = jnp.full((LANES,), i, dtype=jnp.int32)
        rrot = (plsc.load_gather(br0, [spl, perm])
                + plsc.load_gather(br1, [spl, perm]))
        aggbuf[i] = jnp.where(lo, srow + rrot, tail)
        return carry

    lax.fori_loop(0, RPT, comb, 0)
    pltpu.sync_copy(aggbuf, agg_sh.at[pl.ds(z0, RPT)])
    plsc.subcore_barrier()

    for bq in range(GRP):
        pltpu.async_copy(agg_sh.at[idxo_v.at[bq]],
                         gbuf.at[pl.ds(bq * BLK, BLK)], sem_g)

    def group(g, carry):
        off = (g & 1) * (GRP * BLK)
        for _ in range(GRP):
            pltpu.make_async_copy(zeros_hbm.at[pl.ds(0, BLK)],
                                  gbuf.at[pl.ds(0, BLK)], sem_g).wait()

        @pl.when(g >= 1)
        def _():
            for _ in range(GRP):
                pltpu.make_async_copy(zeros_hbm.at[pl.ds(0, BLK)],
                                      gbuf.at[pl.ds(0, BLK)], sem_s).wait()

        @pl.when(g + 1 < NGRP_B)
        def _():
            off2 = (GRP * BLK) - off
            for bq in range(GRP):
                pltpu.async_copy(agg_sh.at[idxo_v.at[(g + 1) * GRP + bq]],
                                 gbuf.at[pl.ds(off2 + bq * BLK, BLK)], sem_g)

        for bq in range(GRP):
            pltpu.async_copy(gbuf.at[pl.ds(off + bq * BLK, BLK)],
                             sums_sh.at[idxi_v.at[g * GRP + bq]],
                             sem_s, add=True)
        return carry

    lax.fori_loop(0, NGRP_B, group, 0)
    for _ in range(GRP):
        pltpu.make_async_copy(zeros_hbm.at[pl.ds(0, BLK)],
                              gbuf.at[pl.ds(0, BLK)], sem_s).wait()
    plsc.subcore_barrier()
    pltpu.sync_copy(sums_sh.at[pl.ds(z0, RPT)], tmp := bs0)
    pltpu.sync_copy(tmp, out_hbm.at[cid, pl.ds(z0, RPT)])


ROWS_BLK = 1000


def _dense_body(x_ref, w_ref, b_ref, out_ref):
    out_ref[...] = jnp.dot(
        x_ref[...], w_ref[HALF:, :],
        preferred_element_type=jnp.float32) + b_ref[...]


def _finish_body(p1_ref, s0_ref, s1_ref, w_ref, out_ref):
    sums = s0_ref[0] + s1_ref[0]
    cnt = jnp.maximum(sums[:, HALF:HALF + 1], 1.0)
    navg = sums[:, :HALF] / cnt
    out_ref[...] = p1_ref[...] + jnp.dot(
        navg, w_ref[:HALF, :], preferred_element_type=jnp.float32)


def kernel(x, edge_attr, W, b, edge_index):
    senders = edge_index[0]
    receivers = edge_index[1]
    sidx = senders.reshape(CT, 128)
    ridx = receivers.reshape(CT, 128)
    ea4 = edge_attr.T.reshape(2, HALF, CT, 128).transpose(0, 2, 1, 3)
    zeros_np = jnp.zeros((NP, ROW_W), dtype=jnp.float32)

    grid = N_NODES // ROWS_BLK
    part1 = pl.pallas_call(
        _dense_body,
        grid=(grid,),
        in_specs=[
            pl.BlockSpec((ROWS_BLK, D_FEAT), lambda i: (i, 0)),
            pl.BlockSpec((D_FEAT + HALF, D_FEAT), lambda i: (0, 0)),
            pl.BlockSpec((1, D_FEAT), lambda i: (0, 0)),
        ],
        out_specs=pl.BlockSpec((ROWS_BLK, D_FEAT), lambda i: (i, 0)),
        out_shape=jax.ShapeDtypeStruct((N_NODES, D_FEAT), jnp.float32),
    )(x, W, b.reshape(1, D_FEAT))

    acc = _scatter_edges(ea4, sidx, ridx, zeros_np)

    idx_in = jnp.concatenate([senders, receivers]).reshape(NBLK_B, BLK)
    idx_out = jnp.concatenate([receivers, senders]).reshape(NBLK_B, BLK)
    sums_pair = _gather_scatter_add(acc, idx_out, idx_in, zeros_np)

    out = pl.pallas_call(
        _finish_body,
        grid=(grid,),
        in_specs=[
            pl.BlockSpec((ROWS_BLK, D_FEAT), lambda i: (i, 0)),
            pl.BlockSpec((1, ROWS_BLK, ROW_W), lambda i: (0, i, 0)),
            pl.BlockSpec((1, ROWS_BLK, ROW_W), lambda i: (1, i, 0)),
            pl.BlockSpec((D_FEAT + HALF, D_FEAT), lambda i: (0, 0)),
        ],
        out_specs=pl.BlockSpec((ROWS_BLK, D_FEAT), lambda i: (i, 0)),
        out_shape=jax.ShapeDtypeStruct((N_NODES, D_FEAT), jnp.float32),
    )(part1, sums_pair, sums_pair, W)
    return out
